# Initial kernel scaffold; baseline (speedup 1.0000x reference)
import jax
import jax.numpy as jnp
from jax.experimental import pallas as pl


def kernel(ref_overlap, src_overlap, ref_overlap_pred, src_overlap_pred, lengths_0, lengths_1, lengths_2, subsampling_0, subsampling_1, ref_indices, src_indices):
    raise NotImplementedError("write your pallas kernel here")


# speedup vs baseline: 148.3040x; 148.3040x over previous
import functools

import jax
import jax.numpy as jnp
from jax import lax
from jax.experimental import pallas as pl
from jax.experimental.pallas import tpu as pltpu
from jax.experimental.pallas import tpu_sc as plsc

NW = 32
LANES = 16

N1_ROWS = 50000
K = 32
INV0 = 100000
GROUPS_W = 98
CH = 14
NCHUNK = 7
ROWS_CHUNK = CH * LANES
PAD_ROWS = NW * GROUPS_W * LANES

N2_ROWS = 25000
INV1 = 50000
M2 = 8192
PW = 256
G2 = PW // LANES

_mesh = plsc.VectorSubcoreMesh(core_axis_name="c", subcore_axis_name="s")
_sc_params = pltpu.CompilerParams(needs_layout_passes=False,
                                  use_tc_tiling_on_sc=False)


def _worker_id():
    return lax.axis_index("s") * 2 + lax.axis_index("c")


@functools.partial(
    pl.kernel,
    mesh=_mesh,
    out_type=jax.ShapeDtypeStruct((PAD_ROWS,), jnp.float32),
    scratch_types=[
        pltpu.VMEM((INV0,), jnp.float32),
        pltpu.VMEM((ROWS_CHUNK * K,), jnp.int32),
        pltpu.VMEM((ROWS_CHUNK,), jnp.float32),
    ],
    compiler_params=_sc_params,
)
def _lvl1(pyr0_hbm, sub0_hbm, out_hbm, table_v, slab_v, outc_v):
    w = _worker_id()
    pltpu.sync_copy(pyr0_hbm, table_v)
    lane32 = lax.broadcasted_iota(jnp.int32, (LANES,), 0) * K

    def chunk_body(cb, carry):
        r0 = (w * GROUPS_W + cb * CH) * LANES
        r0c = jnp.minimum(r0, N1_ROWS - ROWS_CHUNK)
        pltpu.sync_copy(sub0_hbm.at[pl.ds(r0c * K, ROWS_CHUNK * K)], slab_v)

        def group_body(j, carry2):
            base_row = jnp.minimum(r0 - r0c + j * LANES, ROWS_CHUNK - LANES)
            base = base_row * K
            acc = jnp.zeros((LANES,), jnp.float32)
            cnt = jnp.zeros((LANES,), jnp.float32)
            for k in range(K):
                pos = base + k + lane32
                idx = plsc.load_gather(slab_v, [pos])
                valid = idx < INV0
                safe = jnp.where(valid, idx, 0)
                vals = plsc.load_gather(table_v, [safe])
                vf = valid.astype(jnp.float32)
                acc = acc + vals * vf
                cnt = cnt + vf
            pooled = jnp.clip(acc / cnt, 0.0, 1.0)
            outc_v[pl.ds(j * LANES, LANES)] = pooled
            return carry2

        lax.fori_loop(0, CH, group_body, 0)
        pltpu.sync_copy(outc_v, out_hbm.at[pl.ds(r0, ROWS_CHUNK)])
        return carry

    lax.fori_loop(0, NCHUNK, chunk_body, 0)


@functools.partial(
    pl.kernel,
    mesh=_mesh,
    out_type=jax.ShapeDtypeStruct((M2,), jnp.float32),
    scratch_types=[
        pltpu.VMEM((PAD_ROWS,), jnp.float32),
        pltpu.VMEM((2, 128), jnp.int32),
        pltpu.VMEM((PW, K), jnp.int32),
        pltpu.VMEM((PW,), jnp.float32),
        pltpu.SemaphoreType.DMA,
    ],
    compiler_params=_sc_params,
)
def _lvl2(pyr1_hbm, sub1_hbm, rows_hbm, out_hbm, table_v, rows_v, slab_v, out_v, sem):
    w = _worker_id()
    pltpu.sync_copy(pyr1_hbm, table_v)
    pltpu.sync_copy(rows_hbm.at[pl.ds(w * 2, 2)], rows_v)
    for j in range(2):
        pltpu.async_copy(sub1_hbm.at[rows_v.at[j]],
                         slab_v.at[pl.ds(j * 128, 128)], sem).wait()
    lane = lax.broadcasted_iota(jnp.int32, (LANES,), 0)

    def group_body(g, carry):
        row = g * LANES + lane
        acc = jnp.zeros((LANES,), jnp.float32)
        cnt = jnp.zeros((LANES,), jnp.float32)
        for k in range(K):
            col = jnp.full((LANES,), k, jnp.int32)
            idx = plsc.load_gather(slab_v, [row, col])
            valid = idx < INV1
            safe = jnp.where(valid, idx, 0)
            vals = plsc.load_gather(table_v, [safe])
            vf = valid.astype(jnp.float32)
            acc = acc + vals * vf
            cnt = cnt + vf
        out_v[pl.ds(g * LANES, LANES)] = jnp.clip(acc / cnt, 0.0, 1.0)
        return carry

    lax.fori_loop(0, G2, group_body, 0)
    pltpu.sync_copy(out_v, out_hbm.at[pl.ds(w * PW, PW)])


def _bce_body(gt_ref, lg_ref, out_ref):
    gt = gt_ref[...]
    lg = lg_ref[...]
    t = jnp.maximum(lg, 0.0) - lg * gt + jnp.log1p(jnp.exp(-jnp.abs(lg)))
    out_ref[0, 0] = jnp.sum(t) / float(M2)


def kernel(ref_overlap, src_overlap, ref_overlap_pred, src_overlap_pred,
           lengths_0, lengths_1, lengths_2, subsampling_0, subsampling_1,
           ref_indices, src_indices):
    pyr0 = jnp.concatenate([ref_overlap, src_overlap], axis=0).astype(jnp.float32)
    sub0_flat = jnp.reshape(subsampling_0, (-1,))

    pyr1 = _lvl1(pyr0, sub0_flat)

    rows = jnp.concatenate([ref_indices, src_indices + lengths_2[0]], axis=0)
    rows2d = jnp.reshape(rows.astype(jnp.int32), (NW * 2, 128))
    gt = _lvl2(pyr1, subsampling_1, rows2d)

    logits = jnp.concatenate([ref_overlap_pred, src_overlap_pred], axis=-2)[:, 0]
    lg2d = jnp.reshape(logits.astype(jnp.float32), (64, 128))
    gt2d = jnp.reshape(gt, (64, 128))
    loss = pl.pallas_call(
        _bce_body,
        out_shape=jax.ShapeDtypeStruct((1, 1), jnp.float32),
        out_specs=pl.BlockSpec(memory_space=pltpu.SMEM),
    )(gt2d, lg2d)
    return loss[0, 0]
